# baseline (speedup 1.0000x reference)
import jax
import jax.numpy as jnp
from jax import lax
from jax.experimental import pallas as pl
from jax.experimental.pallas import tpu as pltpu
from jax.experimental.pallas import tpu_sc as plsc

_NUM_ENTS = 10000
_NUM_RELS = 200
_H = 128
_Q = 2048
_CH = 50
_NC, _NS = 2, 16
_NW = _NC * _NS
_CHUNK = 64
_ACC_ROWS = 10240
_VPAD = 10240
_RB = 1000
_QB = 256
_VB = 1280


def _sig(x):
    return 1.0 / (1.0 + jnp.exp(-x))


def _make_segsum_ab(epad):
    ept = epad // _NS
    nchunks = ept // _CHUNK
    rows_per = _ACC_ROWS // _NS
    mesh = plsc.VectorSubcoreMesh(core_axis_name="c", subcore_axis_name="s",
                                  num_cores=_NC, num_subcores=_NS)
    assert nchunks % 2 == 0

    def body(src_hbm, dst_hbm, et_hbm, tab_hbm, relt_hbm, out_hbm,
             is0, id0, is1, id1, ra0, ra1, zb_a, sa0, sa1, acc):
        cid = lax.axis_index("c")
        sid = lax.axis_index("s")
        r0 = sid * rows_per
        base = sid * ept
        zeros16 = jnp.zeros((16,), jnp.float32)
        for j in range(16):
            for k in range(_H // 16):
                zb_a[j, pl.ds(k * 16, 16)] = zeros16

        def zstep(i, carry):
            pltpu.sync_copy(zb_a, acc.at[pl.ds(r0 + i * 16, 16)])
            return carry

        lax.fori_loop(0, rows_per // 16, zstep, 0)
        plsc.subcore_barrier()

        def chunk_loop(tab_hbm, ihbm):
            bufs = ((is0, id0, ra0, sa0), (is1, id1, ra1, sa1))

            def issue(c, b):
                s_, d_, ra, sa = bufs[b]
                off = base + c * _CHUNK
                pltpu.sync_copy(ihbm.at[pl.ds(off, _CHUNK)], s_)
                pltpu.sync_copy(dst_hbm.at[pl.ds(off, _CHUNK)], d_)
                pltpu.async_copy(tab_hbm.at[s_], ra, sa)

            def drain(b):
                s_, d_, ra, sa = bufs[b]
                pltpu.make_async_copy(tab_hbm.at[pl.ds(0, _CHUNK)],
                                      ra, sa).wait()
                pltpu.sync_copy(ra, acc.at[d_], add=True)

            issue(0, 0)

            def step(j, carry):
                c = 2 * j
                issue(c + 1, 1)
                drain(0)
                issue(c + 2, 0)
                drain(1)
                return carry

            lax.fori_loop(0, nchunks // 2 - 1, step, 0)
            issue(nchunks - 1, 1)
            drain(0)
            drain(1)

        @pl.when(cid == 0)
        def _():
            chunk_loop(tab_hbm, src_hbm)

        @pl.when(cid == 1)
        def _():
            chunk_loop(relt_hbm, et_hbm)

        plsc.subcore_barrier()
        pltpu.sync_copy(acc.at[pl.ds(r0, rows_per)],
                        out_hbm.at[cid, pl.ds(r0, rows_per)])

    return pl.kernel(
        body,
        out_type=jax.ShapeDtypeStruct((_NC, _ACC_ROWS, _H), jnp.float32),
        mesh=mesh,
        scratch_types=[
            pltpu.VMEM((_CHUNK,), jnp.int32),
            pltpu.VMEM((_CHUNK,), jnp.int32),
            pltpu.VMEM((_CHUNK,), jnp.int32),
            pltpu.VMEM((_CHUNK,), jnp.int32),
            pltpu.VMEM((_CHUNK, _H), jnp.float32),
            pltpu.VMEM((_CHUNK, _H), jnp.float32),
            pltpu.VMEM((16, _H), jnp.float32),
            pltpu.SemaphoreType.DMA,
            pltpu.SemaphoreType.DMA,
            pltpu.VMEM_SHARED((_ACC_ROWS, _H), jnp.float32),
        ],
    )


def _make_segsum_h(epad):
    ept = epad // _NW
    nchunks = ept // _CHUNK
    rows_per = _ACC_ROWS // _NS
    mesh = plsc.VectorSubcoreMesh(core_axis_name="c", subcore_axis_name="s",
                                  num_cores=_NC, num_subcores=_NS)
    assert nchunks % 2 == 0

    def body(src_hbm, dst_hbm, tab_hbm, out_hbm,
             is0, id0, is1, id1, ra0, ra1, zb_a, sa0, sa1, acc):
        cid = lax.axis_index("c")
        sid = lax.axis_index("s")
        wid = cid * _NS + sid
        r0 = sid * rows_per
        base = wid * ept
        bufs = ((is0, id0, ra0, sa0), (is1, id1, ra1, sa1))

        def issue(c, b):
            s_, d_, ra, sa = bufs[b]
            off = base + c * _CHUNK
            pltpu.sync_copy(src_hbm.at[pl.ds(off, _CHUNK)], s_)
            pltpu.sync_copy(dst_hbm.at[pl.ds(off, _CHUNK)], d_)
            pltpu.async_copy(tab_hbm.at[s_], ra, sa)

        def drain(b):
            s_, d_, ra, sa = bufs[b]
            pltpu.make_async_copy(tab_hbm.at[pl.ds(0, _CHUNK)], ra, sa).wait()
            pltpu.sync_copy(ra, acc.at[d_], add=True)

        issue(0, 0)
        zeros16 = jnp.zeros((16,), jnp.float32)
        for j in range(16):
            for k in range(_H // 16):
                zb_a[j, pl.ds(k * 16, 16)] = zeros16

        def zstep(i, carry):
            pltpu.sync_copy(zb_a, acc.at[pl.ds(r0 + i * 16, 16)])
            return carry

        lax.fori_loop(0, rows_per // 16, zstep, 0)
        plsc.subcore_barrier()

        def step(j, carry):
            c = 2 * j
            issue(c + 1, 1)
            drain(0)
            issue(c + 2, 0)
            drain(1)
            return carry

        lax.fori_loop(0, nchunks // 2 - 1, step, 0)
        issue(nchunks - 1, 1)
        drain(0)
        drain(1)
        plsc.subcore_barrier()
        pltpu.sync_copy(acc.at[pl.ds(r0, rows_per)],
                        out_hbm.at[cid, pl.ds(r0, rows_per)])

    return pl.kernel(
        body,
        out_type=jax.ShapeDtypeStruct((_NC, _ACC_ROWS, _H), jnp.float32),
        mesh=mesh,
        scratch_types=[
            pltpu.VMEM((_CHUNK,), jnp.int32),
            pltpu.VMEM((_CHUNK,), jnp.int32),
            pltpu.VMEM((_CHUNK,), jnp.int32),
            pltpu.VMEM((_CHUNK,), jnp.int32),
            pltpu.VMEM((_CHUNK, _H), jnp.float32),
            pltpu.VMEM((_CHUNK, _H), jnp.float32),
            pltpu.VMEM((16, _H), jnp.float32),
            pltpu.SemaphoreType.DMA,
            pltpu.SemaphoreType.DMA,
            pltpu.VMEM_SHARED((_ACC_ROWS, _H), jnp.float32),
        ],
    )


def _make_deg(epad):
    ept = epad // _NW
    nchunks = ept // _CHUNK
    rows_per = _ACC_ROWS // _NS
    mesh = plsc.VectorSubcoreMesh(core_axis_name="c", subcore_axis_name="s",
                                  num_cores=_NC, num_subcores=_NS)

    assert nchunks % 2 == 0

    def body(dst_hbm, out_hbm, id0, id1, ones_b, zb_a, si0, si1, acc):
        cid = lax.axis_index("c")
        sid = lax.axis_index("s")
        wid = cid * _NS + sid
        r0 = sid * rows_per
        base = wid * ept
        bufs = ((id0, si0), (id1, si1))

        def issue(c, b):
            d_, si = bufs[b]
            pltpu.async_copy(dst_hbm.at[pl.ds(base + c * _CHUNK, _CHUNK)],
                             d_, si)

        def drain(b):
            d_, si = bufs[b]
            pltpu.make_async_copy(dst_hbm.at[pl.ds(0, _CHUNK)], d_, si).wait()
            pltpu.sync_copy(ones_b, acc.at[d_], add=True)

        issue(0, 0)
        zeros16 = jnp.zeros((16,), jnp.float32)
        ones16 = jnp.ones((16,), jnp.float32)
        for j in range(16):
            for k in range(_H // 16):
                zb_a[j, pl.ds(k * 16, 16)] = zeros16
        for j in range(_CHUNK):
            for k in range(_H // 16):
                ones_b[j, pl.ds(k * 16, 16)] = ones16

        def zstep(i, carry):
            pltpu.sync_copy(zb_a, acc.at[pl.ds(r0 + i * 16, 16)])
            return carry

        lax.fori_loop(0, rows_per // 16, zstep, 0)
        plsc.subcore_barrier()

        def step(j, carry):
            c = 2 * j
            issue(c + 1, 1)
            drain(0)
            issue(c + 2, 0)
            drain(1)
            return carry

        lax.fori_loop(0, nchunks // 2 - 1, step, 0)
        issue(nchunks - 1, 1)
        drain(0)
        drain(1)
        plsc.subcore_barrier()
        pltpu.sync_copy(acc.at[pl.ds(r0, rows_per)],
                        out_hbm.at[cid, pl.ds(r0, rows_per)])

    return pl.kernel(
        body,
        out_type=jax.ShapeDtypeStruct((_NC, _ACC_ROWS, _H), jnp.float32),
        mesh=mesh,
        scratch_types=[
            pltpu.VMEM((_CHUNK,), jnp.int32),
            pltpu.VMEM((_CHUNK,), jnp.int32),
            pltpu.VMEM((_CHUNK, _H), jnp.float32),
            pltpu.VMEM((16, _H), jnp.float32),
            pltpu.SemaphoreType.DMA,
            pltpu.SemaphoreType.DMA,
            pltpu.VMEM_SHARED((_ACC_ROWS, _H), jnp.float32),
        ],
    )


def _make_gatherq():
    qpt = _Q // _NW
    mesh = plsc.VectorSubcoreMesh(core_axis_name="c", subcore_axis_name="s",
                                  num_cores=_NC, num_subcores=_NS)

    def body(ent_hbm, relh_hbm, subj_hbm, obj_hbm, relq_hbm,
             o1, o2, o3, idxb, rows, sem):
        cid = lax.axis_index("c")
        sid = lax.axis_index("s")
        base = (cid * _NS + sid) * qpt
        for ih, th, oh in ((subj_hbm, ent_hbm, o1),
                           (obj_hbm, ent_hbm, o2),
                           (relq_hbm, relh_hbm, o3)):
            pltpu.sync_copy(ih.at[pl.ds(base, qpt)], idxb)
            pltpu.async_copy(th.at[idxb], rows, sem).wait()
            pltpu.sync_copy(rows, oh.at[pl.ds(base, qpt)])

    return pl.kernel(
        body,
        out_type=[jax.ShapeDtypeStruct((_Q, _H), jnp.float32)] * 3,
        mesh=mesh,
        scratch_types=[
            pltpu.VMEM((qpt,), jnp.int32),
            pltpu.VMEM((qpt, _H), jnp.float32),
            pltpu.SemaphoreType.DMA,
        ],
    )


def _combine1(parts, degp, h_in, wn, ws):
    def body(p_ref, d_ref, h_ref, wn_ref, ws_ref, o_ref):
        s = p_ref[0] + p_ref[1]
        deg = jnp.maximum((d_ref[0] + d_ref[1])[:, 0:1], 1.0)
        acc = jnp.dot(s / deg, wn_ref[...], preferred_element_type=jnp.float32)
        acc = acc + jnp.dot(h_ref[...], ws_ref[...],
                            preferred_element_type=jnp.float32)
        o_ref[...] = jnp.maximum(acc, 0.0)

    return pl.pallas_call(
        body,
        grid=(_NUM_ENTS // _RB,),
        in_specs=[pl.BlockSpec((2, _RB, _H), lambda m: (0, m, 0)),
                  pl.BlockSpec((2, _RB, _H), lambda m: (0, m, 0)),
                  pl.BlockSpec((_RB, _H), lambda m: (m, 0)),
                  pl.BlockSpec((_H, _H), lambda m: (0, 0)),
                  pl.BlockSpec((_H, _H), lambda m: (0, 0))],
        out_specs=pl.BlockSpec((_RB, _H), lambda m: (m, 0)),
        out_shape=jax.ShapeDtypeStruct((_NUM_ENTS, _H), jnp.float32),
    )(parts, degp, h_in, wn, ws)


def _combine2(parts, parts_a, degp, h_in, wn, ws, e0, linT, lin_b):
    def body(p_ref, pa_ref, d_ref, h_ref, wn_ref, ws_ref, e0_ref, lw_ref,
             lb_ref, o_ref):
        s = p_ref[0] + p_ref[1] + pa_ref[0]
        deg = jnp.maximum((d_ref[0] + d_ref[1])[:, 0:1], 1.0)
        acc = jnp.dot(s / deg, wn_ref[...], preferred_element_type=jnp.float32)
        acc = acc + jnp.dot(h_ref[...], ws_ref[...],
                            preferred_element_type=jnp.float32)
        h2 = jnp.maximum(acc, 0.0)
        e0 = e0_ref[...]
        u = _sig(jnp.dot(e0, lw_ref[...], preferred_element_type=jnp.float32)
                 + lb_ref[...])
        o_ref[...] = e0 + u * (h2 - e0)

    return pl.pallas_call(
        body,
        grid=(_NUM_ENTS // _RB,),
        in_specs=[pl.BlockSpec((2, _RB, _H), lambda m: (0, m, 0)),
                  pl.BlockSpec((1, _RB, _H), lambda m: (1, m, 0)),
                  pl.BlockSpec((2, _RB, _H), lambda m: (0, m, 0)),
                  pl.BlockSpec((_RB, _H), lambda m: (m, 0)),
                  pl.BlockSpec((_H, _H), lambda m: (0, 0)),
                  pl.BlockSpec((_H, _H), lambda m: (0, 0)),
                  pl.BlockSpec((_RB, _H), lambda m: (m, 0)),
                  pl.BlockSpec((_H, _H), lambda m: (0, 0)),
                  pl.BlockSpec((1, _H), lambda m: (0, 0))],
        out_specs=pl.BlockSpec((_RB, _H), lambda m: (m, 0)),
        out_shape=jax.ShapeDtypeStruct((_VPAD, _H), jnp.float32),
    )(parts, parts_a, degp, h_in, wn, ws, e0, linT, lin_b)


def _relgru(rel0, wihT, whhT, bih, bhh):
    def body(r_ref, wi_ref, wh_ref, bi_ref, bh_ref, o_ref):
        r0 = r_ref[...]
        wsum = wi_ref[0:_H, :] + wi_ref[_H:2 * _H, :]
        gi = jnp.dot(r0, wsum, preferred_element_type=jnp.float32) + bi_ref[...]
        gh = jnp.dot(r0, wh_ref[...], preferred_element_type=jnp.float32) \
            + bh_ref[...]
        r = _sig(gi[:, :_H] + gh[:, :_H])
        z = _sig(gi[:, _H:2 * _H] + gh[:, _H:2 * _H])
        n = jnp.tanh(gi[:, 2 * _H:] + r * gh[:, 2 * _H:])
        o_ref[...] = (1.0 - z) * n + z * r0

    return pl.pallas_call(
        body,
        out_shape=jax.ShapeDtypeStruct((_NUM_RELS, _H), jnp.float32),
    )(rel0, wihT, whhT, bih, bhh)


def _dec_hidden(e1, e2, cw, cb, fw, fb):
    def body(e1_ref, e2_ref, cw_ref, cb_ref, fw_ref, fb_ref, o_ref):
        e1 = e1_ref[...]
        e2 = e2_ref[...]
        z = jnp.zeros((_QB, 1), jnp.float32)
        u = (jnp.concatenate([z, e1[:, :-1]], axis=1), e1,
             jnp.concatenate([e1[:, 1:], z], axis=1),
             jnp.concatenate([z, e2[:, :-1]], axis=1), e2,
             jnp.concatenate([e2[:, 1:], z], axis=1))
        acc = jnp.zeros((_QB, _H), jnp.float32)
        for c in range(_CH):
            f = cb_ref[c] + u[0] * cw_ref[c, 0] + u[1] * cw_ref[c, 1] \
                + u[2] * cw_ref[c, 2] + u[3] * cw_ref[c, 3] \
                + u[4] * cw_ref[c, 4] + u[5] * cw_ref[c, 5]
            f = jnp.maximum(f, 0.0)
            acc = acc + jnp.dot(f, fw_ref[c],
                                preferred_element_type=jnp.float32)
        o_ref[...] = jnp.maximum(acc + fb_ref[...], 0.0)

    return pl.pallas_call(
        body,
        grid=(_Q // _QB,),
        in_specs=[pl.BlockSpec((_QB, _H), lambda m: (m, 0)),
                  pl.BlockSpec((_QB, _H), lambda m: (m, 0)),
                  pl.BlockSpec(memory_space=pltpu.SMEM),
                  pl.BlockSpec(memory_space=pltpu.SMEM),
                  pl.BlockSpec((_CH, _H, _H), lambda m: (0, 0, 0)),
                  pl.BlockSpec((1, _H), lambda m: (0, 0))],
        out_specs=pl.BlockSpec((_QB, _H), lambda m: (m, 0)),
        out_shape=jax.ShapeDtypeStruct((_Q, _H), jnp.float32),
    )(e1, e2, cw, cb, fw, fb)


def _logits(hid, score, vb):
    nv = score.shape[0] // vb

    def body(h_ref, s_ref, o_ref):
        o_ref[...] = lax.dot_general(
            h_ref[...], s_ref[...], (((1,), (1,)), ((), ())),
            preferred_element_type=jnp.float32)

    return pl.pallas_call(
        body,
        grid=(_Q // _QB, nv),
        in_specs=[pl.BlockSpec((_QB, _H), lambda m, v: (m, 0)),
                  pl.BlockSpec((vb, _H), lambda m, v: (v, 0))],
        out_specs=pl.BlockSpec((_QB, vb), lambda m, v: (m, v)),
        out_shape=jax.ShapeDtypeStruct((_Q, score.shape[0]), jnp.float32),
    )(hid, score)


def kernel(edge_src, edge_dst, edge_type, subj, rel, obj, ent_embeds,
           rel_embeds, rgcn_w_neigh, rgcn_w_self, evo_w_neigh, evo_w_self,
           gru_w_ih, gru_w_hh, gru_b_ih, gru_b_hh, lin_w, lin_b,
           relgru_w_ih, relgru_w_hh, relgru_b_ih, relgru_b_hh,
           convR_w, convR_b, fcR_w, fcR_b, convE_w, convE_b, fcE_w, fcE_b):
    f32, i32 = jnp.float32, jnp.int32
    e = edge_src.shape[1]
    gran = _NW * _CHUNK * 2
    epad = ((e + gran - 1) // gran) * gran
    pad = epad - e
    src0 = jnp.concatenate([edge_src[0].astype(i32),
                            jnp.zeros((pad,), i32)])
    dst0 = jnp.concatenate([edge_dst[0].astype(i32),
                            jnp.full((pad,), _ACC_ROWS - 1, i32)])
    et0 = jnp.concatenate([edge_type[0].astype(i32),
                           jnp.zeros((pad,), i32)])
    ent0 = ent_embeds.astype(f32)
    rel0 = rel_embeds.astype(f32)

    parts_a = _make_segsum_ab(epad)(src0, dst0, et0, ent0, rel0)
    deg_a = _make_deg(epad)(dst0)
    h1 = _combine1(parts_a, deg_a, ent0,
                   evo_w_neigh[0], evo_w_self[0])
    parts_b = _make_segsum_h(epad)(src0, dst0, h1)
    n_ent = _combine2(parts_b, parts_a, deg_a, h1,
                      evo_w_neigh[1], evo_w_self[1], ent0,
                      lin_w.T, lin_b[None, :])
    n_rel = _relgru(rel0, relgru_w_ih.T, relgru_w_hh.T,
                    relgru_b_ih[None, :], relgru_b_hh[None, :])

    e1, e2r, e2e = _make_gatherq()(n_ent, n_rel, subj.astype(i32),
                                   obj.astype(i32), rel.astype(i32))

    fwr = jnp.transpose(fcR_w.reshape(_H, _CH, _H), (1, 2, 0))
    fwe = jnp.transpose(fcE_w.reshape(_H, _CH, _H), (1, 2, 0))
    hid_r = _dec_hidden(e1, e2r, convR_w.reshape(_CH, 6), convR_b, fwr,
                        fcR_b[None, :])
    hid_e = _dec_hidden(e1, e2e, convE_w.reshape(_CH, 6), convE_b, fwe,
                        fcE_b[None, :])

    rel_logit = _logits(hid_r, n_rel, _NUM_RELS)
    obj_logit = _logits(hid_e, n_ent, _VB)[:, :_NUM_ENTS]
    return rel_logit, obj_logit


# speedup vs baseline: 1.0356x; 1.0356x over previous
import jax
import jax.numpy as jnp
from jax import lax
from jax.experimental import pallas as pl
from jax.experimental.pallas import tpu as pltpu
from jax.experimental.pallas import tpu_sc as plsc

_NUM_ENTS = 10000
_NUM_RELS = 200
_H = 128
_Q = 2048
_CH = 50
_NC, _NS = 2, 16
_NW = _NC * _NS
_CHUNK = 128
_ACC_ROWS = 10240
_VPAD = 10240
_RB = 1000
_QB = 256
_VB = 1280


def _sig(x):
    return 1.0 / (1.0 + jnp.exp(-x))


def _make_segsum_ab(epad):
    ept = epad // _NS
    nchunks = ept // _CHUNK
    rows_per = _ACC_ROWS // _NS
    mesh = plsc.VectorSubcoreMesh(core_axis_name="c", subcore_axis_name="s",
                                  num_cores=_NC, num_subcores=_NS)
    assert nchunks % 2 == 0

    def body(src_hbm, dst_hbm, et_hbm, tab_hbm, relt_hbm, out_hbm,
             is0, id0, is1, id1, ra0, ra1, zb_a, sa0, sa1, acc):
        cid = lax.axis_index("c")
        sid = lax.axis_index("s")
        r0 = sid * rows_per
        base = sid * ept
        zeros16 = jnp.zeros((16,), jnp.float32)
        for j in range(16):
            for k in range(_H // 16):
                zb_a[j, pl.ds(k * 16, 16)] = zeros16

        def zstep(i, carry):
            pltpu.sync_copy(zb_a, acc.at[pl.ds(r0 + i * 16, 16)])
            return carry

        lax.fori_loop(0, rows_per // 16, zstep, 0)
        plsc.subcore_barrier()

        def chunk_loop(tab_hbm, ihbm):
            bufs = ((is0, id0, ra0, sa0), (is1, id1, ra1, sa1))

            def issue(c, b):
                s_, d_, ra, sa = bufs[b]
                off = base + c * _CHUNK
                pltpu.sync_copy(ihbm.at[pl.ds(off, _CHUNK)], s_)
                pltpu.sync_copy(dst_hbm.at[pl.ds(off, _CHUNK)], d_)
                pltpu.async_copy(tab_hbm.at[s_], ra, sa)

            def drain(b):
                s_, d_, ra, sa = bufs[b]
                pltpu.make_async_copy(tab_hbm.at[pl.ds(0, _CHUNK)],
                                      ra, sa).wait()
                pltpu.sync_copy(ra, acc.at[d_], add=True)

            issue(0, 0)

            def step(j, carry):
                c = 2 * j
                issue(c + 1, 1)
                drain(0)
                issue(c + 2, 0)
                drain(1)
                return carry

            lax.fori_loop(0, nchunks // 2 - 1, step, 0)
            issue(nchunks - 1, 1)
            drain(0)
            drain(1)

        @pl.when(cid == 0)
        def _():
            chunk_loop(tab_hbm, src_hbm)

        @pl.when(cid == 1)
        def _():
            chunk_loop(relt_hbm, et_hbm)

        plsc.subcore_barrier()
        pltpu.sync_copy(acc.at[pl.ds(r0, rows_per)],
                        out_hbm.at[cid, pl.ds(r0, rows_per)])

    return pl.kernel(
        body,
        out_type=jax.ShapeDtypeStruct((_NC, _ACC_ROWS, _H), jnp.float32),
        mesh=mesh,
        scratch_types=[
            pltpu.VMEM((_CHUNK,), jnp.int32),
            pltpu.VMEM((_CHUNK,), jnp.int32),
            pltpu.VMEM((_CHUNK,), jnp.int32),
            pltpu.VMEM((_CHUNK,), jnp.int32),
            pltpu.VMEM((_CHUNK, _H), jnp.float32),
            pltpu.VMEM((_CHUNK, _H), jnp.float32),
            pltpu.VMEM((16, _H), jnp.float32),
            pltpu.SemaphoreType.DMA,
            pltpu.SemaphoreType.DMA,
            pltpu.VMEM_SHARED((_ACC_ROWS, _H), jnp.float32),
        ],
    )


def _make_segsum_h(epad):
    ept = epad // _NW
    nchunks = ept // _CHUNK
    rows_per = _ACC_ROWS // _NS
    mesh = plsc.VectorSubcoreMesh(core_axis_name="c", subcore_axis_name="s",
                                  num_cores=_NC, num_subcores=_NS)
    assert nchunks % 2 == 0

    def body(src_hbm, dst_hbm, tab_hbm, out_hbm,
             is0, id0, is1, id1, ra0, ra1, zb_a, sa0, sa1, acc):
        cid = lax.axis_index("c")
        sid = lax.axis_index("s")
        wid = cid * _NS + sid
        r0 = sid * rows_per
        base = wid * ept
        bufs = ((is0, id0, ra0, sa0), (is1, id1, ra1, sa1))

        def issue(c, b):
            s_, d_, ra, sa = bufs[b]
            off = base + c * _CHUNK
            pltpu.sync_copy(src_hbm.at[pl.ds(off, _CHUNK)], s_)
            pltpu.sync_copy(dst_hbm.at[pl.ds(off, _CHUNK)], d_)
            pltpu.async_copy(tab_hbm.at[s_], ra, sa)

        def drain(b):
            s_, d_, ra, sa = bufs[b]
            pltpu.make_async_copy(tab_hbm.at[pl.ds(0, _CHUNK)], ra, sa).wait()
            pltpu.sync_copy(ra, acc.at[d_], add=True)

        issue(0, 0)
        zeros16 = jnp.zeros((16,), jnp.float32)
        for j in range(16):
            for k in range(_H // 16):
                zb_a[j, pl.ds(k * 16, 16)] = zeros16

        def zstep(i, carry):
            pltpu.sync_copy(zb_a, acc.at[pl.ds(r0 + i * 16, 16)])
            return carry

        lax.fori_loop(0, rows_per // 16, zstep, 0)
        plsc.subcore_barrier()

        def step(j, carry):
            c = 2 * j
            issue(c + 1, 1)
            drain(0)
            issue(c + 2, 0)
            drain(1)
            return carry

        lax.fori_loop(0, nchunks // 2 - 1, step, 0)
        issue(nchunks - 1, 1)
        drain(0)
        drain(1)
        plsc.subcore_barrier()
        pltpu.sync_copy(acc.at[pl.ds(r0, rows_per)],
                        out_hbm.at[cid, pl.ds(r0, rows_per)])

    return pl.kernel(
        body,
        out_type=jax.ShapeDtypeStruct((_NC, _ACC_ROWS, _H), jnp.float32),
        mesh=mesh,
        scratch_types=[
            pltpu.VMEM((_CHUNK,), jnp.int32),
            pltpu.VMEM((_CHUNK,), jnp.int32),
            pltpu.VMEM((_CHUNK,), jnp.int32),
            pltpu.VMEM((_CHUNK,), jnp.int32),
            pltpu.VMEM((_CHUNK, _H), jnp.float32),
            pltpu.VMEM((_CHUNK, _H), jnp.float32),
            pltpu.VMEM((16, _H), jnp.float32),
            pltpu.SemaphoreType.DMA,
            pltpu.SemaphoreType.DMA,
            pltpu.VMEM_SHARED((_ACC_ROWS, _H), jnp.float32),
        ],
    )


def _make_deg(epad):
    ept = epad // _NW
    nchunks = ept // _CHUNK
    rows_per = _ACC_ROWS // _NS
    mesh = plsc.VectorSubcoreMesh(core_axis_name="c", subcore_axis_name="s",
                                  num_cores=_NC, num_subcores=_NS)

    assert nchunks % 2 == 0

    def body(dst_hbm, out_hbm, id0, id1, ones_b, zb_a, si0, si1, acc):
        cid = lax.axis_index("c")
        sid = lax.axis_index("s")
        wid = cid * _NS + sid
        r0 = sid * rows_per
        base = wid * ept
        bufs = ((id0, si0), (id1, si1))

        def issue(c, b):
            d_, si = bufs[b]
            pltpu.async_copy(dst_hbm.at[pl.ds(base + c * _CHUNK, _CHUNK)],
                             d_, si)

        def drain(b):
            d_, si = bufs[b]
            pltpu.make_async_copy(dst_hbm.at[pl.ds(0, _CHUNK)], d_, si).wait()
            pltpu.sync_copy(ones_b, acc.at[d_], add=True)

        issue(0, 0)
        zeros16 = jnp.zeros((16,), jnp.float32)
        ones16 = jnp.ones((16,), jnp.float32)
        for j in range(16):
            for k in range(_H // 16):
                zb_a[j, pl.ds(k * 16, 16)] = zeros16
        for j in range(_CHUNK):
            for k in range(_H // 16):
                ones_b[j, pl.ds(k * 16, 16)] = ones16

        def zstep(i, carry):
            pltpu.sync_copy(zb_a, acc.at[pl.ds(r0 + i * 16, 16)])
            return carry

        lax.fori_loop(0, rows_per // 16, zstep, 0)
        plsc.subcore_barrier()

        def step(j, carry):
            c = 2 * j
            issue(c + 1, 1)
            drain(0)
            issue(c + 2, 0)
            drain(1)
            return carry

        lax.fori_loop(0, nchunks // 2 - 1, step, 0)
        issue(nchunks - 1, 1)
        drain(0)
        drain(1)
        plsc.subcore_barrier()
        pltpu.sync_copy(acc.at[pl.ds(r0, rows_per)],
                        out_hbm.at[cid, pl.ds(r0, rows_per)])

    return pl.kernel(
        body,
        out_type=jax.ShapeDtypeStruct((_NC, _ACC_ROWS, _H), jnp.float32),
        mesh=mesh,
        scratch_types=[
            pltpu.VMEM((_CHUNK,), jnp.int32),
            pltpu.VMEM((_CHUNK,), jnp.int32),
            pltpu.VMEM((_CHUNK, _H), jnp.float32),
            pltpu.VMEM((16, _H), jnp.float32),
            pltpu.SemaphoreType.DMA,
            pltpu.SemaphoreType.DMA,
            pltpu.VMEM_SHARED((_ACC_ROWS, _H), jnp.float32),
        ],
    )


def _make_gatherq():
    qpt = _Q // _NW
    mesh = plsc.VectorSubcoreMesh(core_axis_name="c", subcore_axis_name="s",
                                  num_cores=_NC, num_subcores=_NS)

    def body(ent_hbm, relh_hbm, subj_hbm, obj_hbm, relq_hbm,
             o1, o2, o3, idxb, rows, sem):
        cid = lax.axis_index("c")
        sid = lax.axis_index("s")
        base = (cid * _NS + sid) * qpt
        for ih, th, oh in ((subj_hbm, ent_hbm, o1),
                           (obj_hbm, ent_hbm, o2),
                           (relq_hbm, relh_hbm, o3)):
            pltpu.sync_copy(ih.at[pl.ds(base, qpt)], idxb)
            pltpu.async_copy(th.at[idxb], rows, sem).wait()
            pltpu.sync_copy(rows, oh.at[pl.ds(base, qpt)])

    return pl.kernel(
        body,
        out_type=[jax.ShapeDtypeStruct((_Q, _H), jnp.float32)] * 3,
        mesh=mesh,
        scratch_types=[
            pltpu.VMEM((qpt,), jnp.int32),
            pltpu.VMEM((qpt, _H), jnp.float32),
            pltpu.SemaphoreType.DMA,
        ],
    )


def _combine1(parts, degp, h_in, wn, ws):
    def body(p_ref, d_ref, h_ref, wn_ref, ws_ref, o_ref):
        s = p_ref[0] + p_ref[1]
        deg = jnp.maximum((d_ref[0] + d_ref[1])[:, 0:1], 1.0)
        acc = jnp.dot(s / deg, wn_ref[...], preferred_element_type=jnp.float32)
        acc = acc + jnp.dot(h_ref[...], ws_ref[...],
                            preferred_element_type=jnp.float32)
        o_ref[...] = jnp.maximum(acc, 0.0)

    return pl.pallas_call(
        body,
        grid=(_NUM_ENTS // _RB,),
        in_specs=[pl.BlockSpec((2, _RB, _H), lambda m: (0, m, 0)),
                  pl.BlockSpec((2, _RB, _H), lambda m: (0, m, 0)),
                  pl.BlockSpec((_RB, _H), lambda m: (m, 0)),
                  pl.BlockSpec((_H, _H), lambda m: (0, 0)),
                  pl.BlockSpec((_H, _H), lambda m: (0, 0))],
        out_specs=pl.BlockSpec((_RB, _H), lambda m: (m, 0)),
        out_shape=jax.ShapeDtypeStruct((_NUM_ENTS, _H), jnp.float32),
    )(parts, degp, h_in, wn, ws)


def _combine2(parts, parts_a, degp, h_in, wn, ws, e0, linT, lin_b):
    def body(p_ref, pa_ref, d_ref, h_ref, wn_ref, ws_ref, e0_ref, lw_ref,
             lb_ref, o_ref):
        s = p_ref[0] + p_ref[1] + pa_ref[0]
        deg = jnp.maximum((d_ref[0] + d_ref[1])[:, 0:1], 1.0)
        acc = jnp.dot(s / deg, wn_ref[...], preferred_element_type=jnp.float32)
        acc = acc + jnp.dot(h_ref[...], ws_ref[...],
                            preferred_element_type=jnp.float32)
        h2 = jnp.maximum(acc, 0.0)
        e0 = e0_ref[...]
        u = _sig(jnp.dot(e0, lw_ref[...], preferred_element_type=jnp.float32)
                 + lb_ref[...])
        o_ref[...] = e0 + u * (h2 - e0)

    return pl.pallas_call(
        body,
        grid=(_NUM_ENTS // _RB,),
        in_specs=[pl.BlockSpec((2, _RB, _H), lambda m: (0, m, 0)),
                  pl.BlockSpec((1, _RB, _H), lambda m: (1, m, 0)),
                  pl.BlockSpec((2, _RB, _H), lambda m: (0, m, 0)),
                  pl.BlockSpec((_RB, _H), lambda m: (m, 0)),
                  pl.BlockSpec((_H, _H), lambda m: (0, 0)),
                  pl.BlockSpec((_H, _H), lambda m: (0, 0)),
                  pl.BlockSpec((_RB, _H), lambda m: (m, 0)),
                  pl.BlockSpec((_H, _H), lambda m: (0, 0)),
                  pl.BlockSpec((1, _H), lambda m: (0, 0))],
        out_specs=pl.BlockSpec((_RB, _H), lambda m: (m, 0)),
        out_shape=jax.ShapeDtypeStruct((_VPAD, _H), jnp.float32),
    )(parts, parts_a, degp, h_in, wn, ws, e0, linT, lin_b)


def _relgru(rel0, wihT, whhT, bih, bhh):
    def body(r_ref, wi_ref, wh_ref, bi_ref, bh_ref, o_ref):
        r0 = r_ref[...]
        wsum = wi_ref[0:_H, :] + wi_ref[_H:2 * _H, :]
        gi = jnp.dot(r0, wsum, preferred_element_type=jnp.float32) + bi_ref[...]
        gh = jnp.dot(r0, wh_ref[...], preferred_element_type=jnp.float32) \
            + bh_ref[...]
        r = _sig(gi[:, :_H] + gh[:, :_H])
        z = _sig(gi[:, _H:2 * _H] + gh[:, _H:2 * _H])
        n = jnp.tanh(gi[:, 2 * _H:] + r * gh[:, 2 * _H:])
        o_ref[...] = (1.0 - z) * n + z * r0

    return pl.pallas_call(
        body,
        out_shape=jax.ShapeDtypeStruct((_NUM_RELS, _H), jnp.float32),
    )(rel0, wihT, whhT, bih, bhh)


def _dec_hidden(e1, e2, cw, cb, fw, fb):
    def body(e1_ref, e2_ref, cw_ref, cb_ref, fw_ref, fb_ref, o_ref):
        e1 = e1_ref[...]
        e2 = e2_ref[...]
        z = jnp.zeros((_QB, 1), jnp.float32)
        u = (jnp.concatenate([z, e1[:, :-1]], axis=1), e1,
             jnp.concatenate([e1[:, 1:], z], axis=1),
             jnp.concatenate([z, e2[:, :-1]], axis=1), e2,
             jnp.concatenate([e2[:, 1:], z], axis=1))
        acc = jnp.zeros((_QB, _H), jnp.float32)
        for c in range(_CH):
            f = cb_ref[c] + u[0] * cw_ref[c, 0] + u[1] * cw_ref[c, 1] \
                + u[2] * cw_ref[c, 2] + u[3] * cw_ref[c, 3] \
                + u[4] * cw_ref[c, 4] + u[5] * cw_ref[c, 5]
            f = jnp.maximum(f, 0.0)
            acc = acc + jnp.dot(f, fw_ref[c],
                                preferred_element_type=jnp.float32)
        o_ref[...] = jnp.maximum(acc + fb_ref[...], 0.0)

    return pl.pallas_call(
        body,
        grid=(_Q // _QB,),
        in_specs=[pl.BlockSpec((_QB, _H), lambda m: (m, 0)),
                  pl.BlockSpec((_QB, _H), lambda m: (m, 0)),
                  pl.BlockSpec(memory_space=pltpu.SMEM),
                  pl.BlockSpec(memory_space=pltpu.SMEM),
                  pl.BlockSpec((_CH, _H, _H), lambda m: (0, 0, 0)),
                  pl.BlockSpec((1, _H), lambda m: (0, 0))],
        out_specs=pl.BlockSpec((_QB, _H), lambda m: (m, 0)),
        out_shape=jax.ShapeDtypeStruct((_Q, _H), jnp.float32),
    )(e1, e2, cw, cb, fw, fb)


def _logits(hid, score, vb):
    nv = score.shape[0] // vb

    def body(h_ref, s_ref, o_ref):
        o_ref[...] = lax.dot_general(
            h_ref[...], s_ref[...], (((1,), (1,)), ((), ())),
            preferred_element_type=jnp.float32)

    return pl.pallas_call(
        body,
        grid=(_Q // _QB, nv),
        in_specs=[pl.BlockSpec((_QB, _H), lambda m, v: (m, 0)),
                  pl.BlockSpec((vb, _H), lambda m, v: (v, 0))],
        out_specs=pl.BlockSpec((_QB, vb), lambda m, v: (m, v)),
        out_shape=jax.ShapeDtypeStruct((_Q, score.shape[0]), jnp.float32),
    )(hid, score)


def kernel(edge_src, edge_dst, edge_type, subj, rel, obj, ent_embeds,
           rel_embeds, rgcn_w_neigh, rgcn_w_self, evo_w_neigh, evo_w_self,
           gru_w_ih, gru_w_hh, gru_b_ih, gru_b_hh, lin_w, lin_b,
           relgru_w_ih, relgru_w_hh, relgru_b_ih, relgru_b_hh,
           convR_w, convR_b, fcR_w, fcR_b, convE_w, convE_b, fcE_w, fcE_b):
    f32, i32 = jnp.float32, jnp.int32
    e = edge_src.shape[1]
    gran = _NW * _CHUNK * 2
    epad = ((e + gran - 1) // gran) * gran
    pad = epad - e
    src0 = jnp.concatenate([edge_src[0].astype(i32),
                            jnp.zeros((pad,), i32)])
    dst0 = jnp.concatenate([edge_dst[0].astype(i32),
                            jnp.full((pad,), _ACC_ROWS - 1, i32)])
    et0 = jnp.concatenate([edge_type[0].astype(i32),
                           jnp.zeros((pad,), i32)])
    ent0 = ent_embeds.astype(f32)
    rel0 = rel_embeds.astype(f32)

    parts_a = _make_segsum_ab(epad)(src0, dst0, et0, ent0, rel0)
    deg_a = _make_deg(epad)(dst0)
    h1 = _combine1(parts_a, deg_a, ent0,
                   evo_w_neigh[0], evo_w_self[0])
    parts_b = _make_segsum_h(epad)(src0, dst0, h1)
    n_ent = _combine2(parts_b, parts_a, deg_a, h1,
                      evo_w_neigh[1], evo_w_self[1], ent0,
                      lin_w.T, lin_b[None, :])
    n_rel = _relgru(rel0, relgru_w_ih.T, relgru_w_hh.T,
                    relgru_b_ih[None, :], relgru_b_hh[None, :])

    e1, e2r, e2e = _make_gatherq()(n_ent, n_rel, subj.astype(i32),
                                   obj.astype(i32), rel.astype(i32))

    fwr = jnp.transpose(fcR_w.reshape(_H, _CH, _H), (1, 2, 0))
    fwe = jnp.transpose(fcE_w.reshape(_H, _CH, _H), (1, 2, 0))
    hid_r = _dec_hidden(e1, e2r, convR_w.reshape(_CH, 6), convR_b, fwr,
                        fcR_b[None, :])
    hid_e = _dec_hidden(e1, e2e, convE_w.reshape(_CH, 6), convE_b, fwe,
                        fcE_b[None, :])

    rel_logit = _logits(hid_r, n_rel, _NUM_RELS)
    obj_logit = _logits(hid_e, n_ent, _VB)[:, :_NUM_ENTS]
    return rel_logit, obj_logit


# speedup vs baseline: 1.0371x; 1.0015x over previous
import jax
import jax.numpy as jnp
from jax import lax
from jax.experimental import pallas as pl
from jax.experimental.pallas import tpu as pltpu
from jax.experimental.pallas import tpu_sc as plsc

_NUM_ENTS = 10000
_NUM_RELS = 200
_H = 128
_Q = 2048
_CH = 50
_NC, _NS = 2, 16
_NW = _NC * _NS
_CHUNK = 128
_ACC_ROWS = 10240
_VPAD = 10240
_RB = 1000
_QB = 256
_VB = 1280


def _sig(x):
    return 1.0 / (1.0 + jnp.exp(-x))


def _make_segsum_ab(epad):
    ept = epad // _NS
    nchunks = ept // _CHUNK
    rows_per = _ACC_ROWS // _NS
    mesh = plsc.VectorSubcoreMesh(core_axis_name="c", subcore_axis_name="s",
                                  num_cores=_NC, num_subcores=_NS)
    assert nchunks % 2 == 0

    def body(src_hbm, dst_hbm, et_hbm, tab_hbm, relt_hbm, out_hbm,
             is0, id0, is1, id1, ra0, ra1, zb_a, sa0, sa1, acc, rel_spm):
        cid = lax.axis_index("c")
        sid = lax.axis_index("s")
        r0 = sid * rows_per
        base = sid * ept

        @pl.when(sid == 0)
        def _():
            pltpu.sync_copy(relt_hbm, rel_spm)

        zeros16 = jnp.zeros((16,), jnp.float32)
        for j in range(16):
            for k in range(_H // 16):
                zb_a[j, pl.ds(k * 16, 16)] = zeros16

        def zstep(i, carry):
            pltpu.sync_copy(zb_a, acc.at[pl.ds(r0 + i * 16, 16)])
            return carry

        lax.fori_loop(0, rows_per // 16, zstep, 0)
        plsc.subcore_barrier()

        def chunk_loop(tab, ihbm):
            bufs = ((is0, id0, ra0, sa0), (is1, id1, ra1, sa1))

            def issue(c, b):
                s_, d_, ra, sa = bufs[b]
                off = base + c * _CHUNK
                pltpu.sync_copy(ihbm.at[pl.ds(off, _CHUNK)], s_)
                pltpu.sync_copy(dst_hbm.at[pl.ds(off, _CHUNK)], d_)
                pltpu.async_copy(tab.at[s_], ra, sa)

            def drain(b):
                s_, d_, ra, sa = bufs[b]
                pltpu.make_async_copy(tab_hbm.at[pl.ds(0, _CHUNK)],
                                      ra, sa).wait()
                pltpu.sync_copy(ra, acc.at[d_], add=True)

            issue(0, 0)

            def step(j, carry):
                c = 2 * j
                issue(c + 1, 1)
                drain(0)
                issue(c + 2, 0)
                drain(1)
                return carry

            lax.fori_loop(0, nchunks // 2 - 1, step, 0)
            issue(nchunks - 1, 1)
            drain(0)
            drain(1)

        @pl.when(cid == 0)
        def _():
            chunk_loop(tab_hbm, src_hbm)

        @pl.when(cid == 1)
        def _():
            chunk_loop(rel_spm, et_hbm)

        plsc.subcore_barrier()
        pltpu.sync_copy(acc.at[pl.ds(r0, rows_per)],
                        out_hbm.at[cid, pl.ds(r0, rows_per)])

    return pl.kernel(
        body,
        out_type=jax.ShapeDtypeStruct((_NC, _ACC_ROWS, _H), jnp.float32),
        mesh=mesh,
        scratch_types=[
            pltpu.VMEM((_CHUNK,), jnp.int32),
            pltpu.VMEM((_CHUNK,), jnp.int32),
            pltpu.VMEM((_CHUNK,), jnp.int32),
            pltpu.VMEM((_CHUNK,), jnp.int32),
            pltpu.VMEM((_CHUNK, _H), jnp.float32),
            pltpu.VMEM((_CHUNK, _H), jnp.float32),
            pltpu.VMEM((16, _H), jnp.float32),
            pltpu.SemaphoreType.DMA,
            pltpu.SemaphoreType.DMA,
            pltpu.VMEM_SHARED((_ACC_ROWS, _H), jnp.float32),
            pltpu.VMEM_SHARED((_NUM_RELS, _H), jnp.float32),
        ],
    )


def _make_segsum_h(epad):
    ept = epad // _NW
    nchunks = ept // _CHUNK
    rows_per = _ACC_ROWS // _NS
    mesh = plsc.VectorSubcoreMesh(core_axis_name="c", subcore_axis_name="s",
                                  num_cores=_NC, num_subcores=_NS)
    assert nchunks % 2 == 0

    def body(src_hbm, dst_hbm, tab_hbm, out_hbm,
             is0, id0, is1, id1, ra0, ra1, zb_a, sa0, sa1, acc):
        cid = lax.axis_index("c")
        sid = lax.axis_index("s")
        wid = cid * _NS + sid
        r0 = sid * rows_per
        base = wid * ept
        bufs = ((is0, id0, ra0, sa0), (is1, id1, ra1, sa1))

        def issue(c, b):
            s_, d_, ra, sa = bufs[b]
            off = base + c * _CHUNK
            pltpu.sync_copy(src_hbm.at[pl.ds(off, _CHUNK)], s_)
            pltpu.sync_copy(dst_hbm.at[pl.ds(off, _CHUNK)], d_)
            pltpu.async_copy(tab_hbm.at[s_], ra, sa)

        def drain(b):
            s_, d_, ra, sa = bufs[b]
            pltpu.make_async_copy(tab_hbm.at[pl.ds(0, _CHUNK)], ra, sa).wait()
            pltpu.sync_copy(ra, acc.at[d_], add=True)

        issue(0, 0)
        zeros16 = jnp.zeros((16,), jnp.float32)
        for j in range(16):
            for k in range(_H // 16):
                zb_a[j, pl.ds(k * 16, 16)] = zeros16

        def zstep(i, carry):
            pltpu.sync_copy(zb_a, acc.at[pl.ds(r0 + i * 16, 16)])
            return carry

        lax.fori_loop(0, rows_per // 16, zstep, 0)
        plsc.subcore_barrier()

        def step(j, carry):
            c = 2 * j
            issue(c + 1, 1)
            drain(0)
            issue(c + 2, 0)
            drain(1)
            return carry

        lax.fori_loop(0, nchunks // 2 - 1, step, 0)
        issue(nchunks - 1, 1)
        drain(0)
        drain(1)
        plsc.subcore_barrier()
        pltpu.sync_copy(acc.at[pl.ds(r0, rows_per)],
                        out_hbm.at[cid, pl.ds(r0, rows_per)])

    return pl.kernel(
        body,
        out_type=jax.ShapeDtypeStruct((_NC, _ACC_ROWS, _H), jnp.float32),
        mesh=mesh,
        scratch_types=[
            pltpu.VMEM((_CHUNK,), jnp.int32),
            pltpu.VMEM((_CHUNK,), jnp.int32),
            pltpu.VMEM((_CHUNK,), jnp.int32),
            pltpu.VMEM((_CHUNK,), jnp.int32),
            pltpu.VMEM((_CHUNK, _H), jnp.float32),
            pltpu.VMEM((_CHUNK, _H), jnp.float32),
            pltpu.VMEM((16, _H), jnp.float32),
            pltpu.SemaphoreType.DMA,
            pltpu.SemaphoreType.DMA,
            pltpu.VMEM_SHARED((_ACC_ROWS, _H), jnp.float32),
        ],
    )


def _make_deg(epad):
    ept = epad // _NW
    nchunks = ept // _CHUNK
    rows_per = _ACC_ROWS // _NS
    mesh = plsc.VectorSubcoreMesh(core_axis_name="c", subcore_axis_name="s",
                                  num_cores=_NC, num_subcores=_NS)

    assert nchunks % 2 == 0

    def body(dst_hbm, out_hbm, id0, id1, ones_b, zb_a, si0, si1, acc):
        cid = lax.axis_index("c")
        sid = lax.axis_index("s")
        wid = cid * _NS + sid
        r0 = sid * rows_per
        base = wid * ept
        bufs = ((id0, si0), (id1, si1))

        def issue(c, b):
            d_, si = bufs[b]
            pltpu.async_copy(dst_hbm.at[pl.ds(base + c * _CHUNK, _CHUNK)],
                             d_, si)

        def drain(b):
            d_, si = bufs[b]
            pltpu.make_async_copy(dst_hbm.at[pl.ds(0, _CHUNK)], d_, si).wait()
            pltpu.sync_copy(ones_b, acc.at[d_], add=True)

        issue(0, 0)
        zeros16 = jnp.zeros((16,), jnp.float32)
        ones16 = jnp.ones((16,), jnp.float32)
        for j in range(16):
            for k in range(_H // 16):
                zb_a[j, pl.ds(k * 16, 16)] = zeros16
        for j in range(_CHUNK):
            for k in range(_H // 16):
                ones_b[j, pl.ds(k * 16, 16)] = ones16

        def zstep(i, carry):
            pltpu.sync_copy(zb_a, acc.at[pl.ds(r0 + i * 16, 16)])
            return carry

        lax.fori_loop(0, rows_per // 16, zstep, 0)
        plsc.subcore_barrier()

        def step(j, carry):
            c = 2 * j
            issue(c + 1, 1)
            drain(0)
            issue(c + 2, 0)
            drain(1)
            return carry

        lax.fori_loop(0, nchunks // 2 - 1, step, 0)
        issue(nchunks - 1, 1)
        drain(0)
        drain(1)
        plsc.subcore_barrier()
        pltpu.sync_copy(acc.at[pl.ds(r0, rows_per)],
                        out_hbm.at[cid, pl.ds(r0, rows_per)])

    return pl.kernel(
        body,
        out_type=jax.ShapeDtypeStruct((_NC, _ACC_ROWS, _H), jnp.float32),
        mesh=mesh,
        scratch_types=[
            pltpu.VMEM((_CHUNK,), jnp.int32),
            pltpu.VMEM((_CHUNK,), jnp.int32),
            pltpu.VMEM((_CHUNK, _H), jnp.float32),
            pltpu.VMEM((16, _H), jnp.float32),
            pltpu.SemaphoreType.DMA,
            pltpu.SemaphoreType.DMA,
            pltpu.VMEM_SHARED((_ACC_ROWS, _H), jnp.float32),
        ],
    )


def _make_gatherq():
    qpt = _Q // _NW
    mesh = plsc.VectorSubcoreMesh(core_axis_name="c", subcore_axis_name="s",
                                  num_cores=_NC, num_subcores=_NS)

    def body(ent_hbm, relh_hbm, subj_hbm, obj_hbm, relq_hbm,
             o1, o2, o3, idxb, rows, sem):
        cid = lax.axis_index("c")
        sid = lax.axis_index("s")
        base = (cid * _NS + sid) * qpt
        for ih, th, oh in ((subj_hbm, ent_hbm, o1),
                           (obj_hbm, ent_hbm, o2),
                           (relq_hbm, relh_hbm, o3)):
            pltpu.sync_copy(ih.at[pl.ds(base, qpt)], idxb)
            pltpu.async_copy(th.at[idxb], rows, sem).wait()
            pltpu.sync_copy(rows, oh.at[pl.ds(base, qpt)])

    return pl.kernel(
        body,
        out_type=[jax.ShapeDtypeStruct((_Q, _H), jnp.float32)] * 3,
        mesh=mesh,
        scratch_types=[
            pltpu.VMEM((qpt,), jnp.int32),
            pltpu.VMEM((qpt, _H), jnp.float32),
            pltpu.SemaphoreType.DMA,
        ],
    )


def _combine1(parts, degp, h_in, wn, ws):
    def body(p_ref, d_ref, h_ref, wn_ref, ws_ref, o_ref):
        s = p_ref[0] + p_ref[1]
        deg = jnp.maximum((d_ref[0] + d_ref[1])[:, 0:1], 1.0)
        acc = jnp.dot(s / deg, wn_ref[...], preferred_element_type=jnp.float32)
        acc = acc + jnp.dot(h_ref[...], ws_ref[...],
                            preferred_element_type=jnp.float32)
        o_ref[...] = jnp.maximum(acc, 0.0)

    return pl.pallas_call(
        body,
        grid=(_NUM_ENTS // _RB,),
        in_specs=[pl.BlockSpec((2, _RB, _H), lambda m: (0, m, 0)),
                  pl.BlockSpec((2, _RB, _H), lambda m: (0, m, 0)),
                  pl.BlockSpec((_RB, _H), lambda m: (m, 0)),
                  pl.BlockSpec((_H, _H), lambda m: (0, 0)),
                  pl.BlockSpec((_H, _H), lambda m: (0, 0))],
        out_specs=pl.BlockSpec((_RB, _H), lambda m: (m, 0)),
        out_shape=jax.ShapeDtypeStruct((_NUM_ENTS, _H), jnp.float32),
    )(parts, degp, h_in, wn, ws)


def _combine2(parts, parts_a, degp, h_in, wn, ws, e0, linT, lin_b):
    def body(p_ref, pa_ref, d_ref, h_ref, wn_ref, ws_ref, e0_ref, lw_ref,
             lb_ref, o_ref):
        s = p_ref[0] + p_ref[1] + pa_ref[0]
        deg = jnp.maximum((d_ref[0] + d_ref[1])[:, 0:1], 1.0)
        acc = jnp.dot(s / deg, wn_ref[...], preferred_element_type=jnp.float32)
        acc = acc + jnp.dot(h_ref[...], ws_ref[...],
                            preferred_element_type=jnp.float32)
        h2 = jnp.maximum(acc, 0.0)
        e0 = e0_ref[...]
        u = _sig(jnp.dot(e0, lw_ref[...], preferred_element_type=jnp.float32)
                 + lb_ref[...])
        o_ref[...] = e0 + u * (h2 - e0)

    return pl.pallas_call(
        body,
        grid=(_NUM_ENTS // _RB,),
        in_specs=[pl.BlockSpec((2, _RB, _H), lambda m: (0, m, 0)),
                  pl.BlockSpec((1, _RB, _H), lambda m: (1, m, 0)),
                  pl.BlockSpec((2, _RB, _H), lambda m: (0, m, 0)),
                  pl.BlockSpec((_RB, _H), lambda m: (m, 0)),
                  pl.BlockSpec((_H, _H), lambda m: (0, 0)),
                  pl.BlockSpec((_H, _H), lambda m: (0, 0)),
                  pl.BlockSpec((_RB, _H), lambda m: (m, 0)),
                  pl.BlockSpec((_H, _H), lambda m: (0, 0)),
                  pl.BlockSpec((1, _H), lambda m: (0, 0))],
        out_specs=pl.BlockSpec((_RB, _H), lambda m: (m, 0)),
        out_shape=jax.ShapeDtypeStruct((_VPAD, _H), jnp.float32),
    )(parts, parts_a, degp, h_in, wn, ws, e0, linT, lin_b)


def _relgru(rel0, wihT, whhT, bih, bhh):
    def body(r_ref, wi_ref, wh_ref, bi_ref, bh_ref, o_ref):
        r0 = r_ref[...]
        wsum = wi_ref[0:_H, :] + wi_ref[_H:2 * _H, :]
        gi = jnp.dot(r0, wsum, preferred_element_type=jnp.float32) + bi_ref[...]
        gh = jnp.dot(r0, wh_ref[...], preferred_element_type=jnp.float32) \
            + bh_ref[...]
        r = _sig(gi[:, :_H] + gh[:, :_H])
        z = _sig(gi[:, _H:2 * _H] + gh[:, _H:2 * _H])
        n = jnp.tanh(gi[:, 2 * _H:] + r * gh[:, 2 * _H:])
        o_ref[...] = (1.0 - z) * n + z * r0

    return pl.pallas_call(
        body,
        out_shape=jax.ShapeDtypeStruct((_NUM_RELS, _H), jnp.float32),
    )(rel0, wihT, whhT, bih, bhh)


def _dec_hidden(e1, e2, cw, cb, fw, fb):
    def body(e1_ref, e2_ref, cw_ref, cb_ref, fw_ref, fb_ref, o_ref):
        e1 = e1_ref[...]
        e2 = e2_ref[...]
        z = jnp.zeros((_QB, 1), jnp.float32)
        u = (jnp.concatenate([z, e1[:, :-1]], axis=1), e1,
             jnp.concatenate([e1[:, 1:], z], axis=1),
             jnp.concatenate([z, e2[:, :-1]], axis=1), e2,
             jnp.concatenate([e2[:, 1:], z], axis=1))
        acc = jnp.zeros((_QB, _H), jnp.float32)
        for c in range(_CH):
            f = cb_ref[c] + u[0] * cw_ref[c, 0] + u[1] * cw_ref[c, 1] \
                + u[2] * cw_ref[c, 2] + u[3] * cw_ref[c, 3] \
                + u[4] * cw_ref[c, 4] + u[5] * cw_ref[c, 5]
            f = jnp.maximum(f, 0.0)
            acc = acc + jnp.dot(f, fw_ref[c],
                                preferred_element_type=jnp.float32)
        o_ref[...] = jnp.maximum(acc + fb_ref[...], 0.0)

    return pl.pallas_call(
        body,
        grid=(_Q // _QB,),
        in_specs=[pl.BlockSpec((_QB, _H), lambda m: (m, 0)),
                  pl.BlockSpec((_QB, _H), lambda m: (m, 0)),
                  pl.BlockSpec(memory_space=pltpu.SMEM),
                  pl.BlockSpec(memory_space=pltpu.SMEM),
                  pl.BlockSpec((_CH, _H, _H), lambda m: (0, 0, 0)),
                  pl.BlockSpec((1, _H), lambda m: (0, 0))],
        out_specs=pl.BlockSpec((_QB, _H), lambda m: (m, 0)),
        out_shape=jax.ShapeDtypeStruct((_Q, _H), jnp.float32),
    )(e1, e2, cw, cb, fw, fb)


def _logits(hid, score, vb):
    nv = score.shape[0] // vb

    def body(h_ref, s_ref, o_ref):
        o_ref[...] = lax.dot_general(
            h_ref[...], s_ref[...], (((1,), (1,)), ((), ())),
            preferred_element_type=jnp.float32)

    return pl.pallas_call(
        body,
        grid=(_Q // _QB, nv),
        in_specs=[pl.BlockSpec((_QB, _H), lambda m, v: (m, 0)),
                  pl.BlockSpec((vb, _H), lambda m, v: (v, 0))],
        out_specs=pl.BlockSpec((_QB, vb), lambda m, v: (m, v)),
        out_shape=jax.ShapeDtypeStruct((_Q, score.shape[0]), jnp.float32),
    )(hid, score)


def kernel(edge_src, edge_dst, edge_type, subj, rel, obj, ent_embeds,
           rel_embeds, rgcn_w_neigh, rgcn_w_self, evo_w_neigh, evo_w_self,
           gru_w_ih, gru_w_hh, gru_b_ih, gru_b_hh, lin_w, lin_b,
           relgru_w_ih, relgru_w_hh, relgru_b_ih, relgru_b_hh,
           convR_w, convR_b, fcR_w, fcR_b, convE_w, convE_b, fcE_w, fcE_b):
    f32, i32 = jnp.float32, jnp.int32
    e = edge_src.shape[1]
    gran = _NW * _CHUNK * 2
    epad = ((e + gran - 1) // gran) * gran
    pad = epad - e
    src0 = jnp.concatenate([edge_src[0].astype(i32),
                            jnp.zeros((pad,), i32)])
    dst0 = jnp.concatenate([edge_dst[0].astype(i32),
                            jnp.full((pad,), _ACC_ROWS - 1, i32)])
    et0 = jnp.concatenate([edge_type[0].astype(i32),
                           jnp.zeros((pad,), i32)])
    ent0 = ent_embeds.astype(f32)
    rel0 = rel_embeds.astype(f32)

    parts_a = _make_segsum_ab(epad)(src0, dst0, et0, ent0, rel0)
    deg_a = _make_deg(epad)(dst0)
    h1 = _combine1(parts_a, deg_a, ent0,
                   evo_w_neigh[0], evo_w_self[0])
    parts_b = _make_segsum_h(epad)(src0, dst0, h1)
    n_ent = _combine2(parts_b, parts_a, deg_a, h1,
                      evo_w_neigh[1], evo_w_self[1], ent0,
                      lin_w.T, lin_b[None, :])
    n_rel = _relgru(rel0, relgru_w_ih.T, relgru_w_hh.T,
                    relgru_b_ih[None, :], relgru_b_hh[None, :])

    e1, e2r, e2e = _make_gatherq()(n_ent, n_rel, subj.astype(i32),
                                   obj.astype(i32), rel.astype(i32))

    fwr = jnp.transpose(fcR_w.reshape(_H, _CH, _H), (1, 2, 0))
    fwe = jnp.transpose(fcE_w.reshape(_H, _CH, _H), (1, 2, 0))
    hid_r = _dec_hidden(e1, e2r, convR_w.reshape(_CH, 6), convR_b, fwr,
                        fcR_b[None, :])
    hid_e = _dec_hidden(e1, e2e, convE_w.reshape(_CH, 6), convE_b, fwe,
                        fcE_b[None, :])

    rel_logit = _logits(hid_r, n_rel, _NUM_RELS)
    obj_logit = _logits(hid_e, n_ent, _VB)[:, :_NUM_ENTS]
    return rel_logit, obj_logit
